# Initial kernel scaffold; baseline (speedup 1.0000x reference)
import jax
import jax.numpy as jnp
from jax.experimental import pallas as pl


def kernel(x, edge_index, batch, ptr, emb, W1, b1, g1, be1, W2, b2, eps, gbn, bbn, Wc1, bc1, Wc2, bc2, Wc3, bc3):
    raise NotImplementedError("write your pallas kernel here")


# speedup vs baseline: 8.2438x; 8.2438x over previous
import functools

import jax
import jax.numpy as jnp
from jax import lax
from jax.experimental import pallas as pl
from jax.experimental.pallas import tpu as pltpu
from jax.experimental.pallas import tpu_sc as plsc

N = 10000
E = 320000
D = 128
L = 3
G = 256
C = 10

NC = 2
NS = 16
NW = NC * NS

EPW = E // NW
K = 128
NFULL = EPW // K
TAIL = EPW - NFULL * K
NP = 10240
STRIPE = NP // NS
EPAD = 128

B = 1000
NB = N // B

_HIGH = jax.lax.Precision.HIGHEST


def _sc_agg_body(hr_hbm, row_hbm, col_hbm, zer_hbm, out0_hbm, out1_hbm,
                 r0, r1, c0, c1, m0, m1, rt, ct, mt, aggsh, sem0, sem1):
    c = lax.axis_index("c")
    s = lax.axis_index("s")
    wid = c * NS + s
    ebase = wid * EPW

    pltpu.sync_copy(zer_hbm, aggsh.at[pl.ds(s * STRIPE, STRIPE)])
    plsc.subcore_barrier()

    pltpu.sync_copy(row_hbm.at[pl.ds(ebase, K)], r0)
    pltpu.sync_copy(col_hbm.at[pl.ds(ebase, K)], c0)
    pltpu.async_copy(hr_hbm.at[r0], m0, sem0)

    def body(g, carry):
        b1 = ebase + (2 * g + 1) * K
        pltpu.sync_copy(row_hbm.at[pl.ds(b1, K)], r1)
        pltpu.sync_copy(col_hbm.at[pl.ds(b1, K)], c1)
        pltpu.async_copy(hr_hbm.at[r1], m1, sem1)
        pltpu.make_async_copy(hr_hbm.at[r0], m0, sem0).wait()
        pltpu.sync_copy(m0, aggsh.at[c0], add=True)
        b2 = ebase + (2 * g + 2) * K
        pltpu.sync_copy(row_hbm.at[pl.ds(b2, K)], r0)
        pltpu.sync_copy(col_hbm.at[pl.ds(b2, K)], c0)
        pltpu.async_copy(hr_hbm.at[r0], m0, sem0)
        pltpu.make_async_copy(hr_hbm.at[r1], m1, sem1).wait()
        pltpu.sync_copy(m1, aggsh.at[c1], add=True)
        return carry

    lax.fori_loop(0, NFULL // 2, body, 0, unroll=False)
    pltpu.make_async_copy(hr_hbm.at[r0], m0, sem0).wait()

    tbase = ebase + NFULL * K
    pltpu.sync_copy(row_hbm.at[pl.ds(tbase, TAIL)], rt)
    pltpu.sync_copy(col_hbm.at[pl.ds(tbase, TAIL)], ct)
    pltpu.async_copy(hr_hbm.at[rt], mt, sem0).wait()
    pltpu.sync_copy(mt, aggsh.at[ct], add=True)

    plsc.subcore_barrier()

    @pl.when(c == 0)
    def _():
        pltpu.sync_copy(aggsh.at[pl.ds(s * STRIPE, STRIPE)],
                        out0_hbm.at[pl.ds(s * STRIPE, STRIPE)])

    @pl.when(c == 1)
    def _():
        pltpu.sync_copy(aggsh.at[pl.ds(s * STRIPE, STRIPE)],
                        out1_hbm.at[pl.ds(s * STRIPE, STRIPE)])


_sc_agg = functools.partial(
    pl.kernel,
    out_type=(jax.ShapeDtypeStruct((NP, D), jnp.float32),
              jax.ShapeDtypeStruct((NP, D), jnp.float32)),
    mesh=plsc.VectorSubcoreMesh(core_axis_name="c", subcore_axis_name="s"),
    scratch_types=[
        pltpu.VMEM((K,), jnp.int32),
        pltpu.VMEM((K,), jnp.int32),
        pltpu.VMEM((K,), jnp.int32),
        pltpu.VMEM((K,), jnp.int32),
        pltpu.VMEM((K, D), jnp.float32),
        pltpu.VMEM((K, D), jnp.float32),
        pltpu.VMEM((TAIL,), jnp.int32),
        pltpu.VMEM((TAIL,), jnp.int32),
        pltpu.VMEM((TAIL, D), jnp.float32),
        pltpu.VMEM_SHARED((NP, D), jnp.float32),
        pltpu.SemaphoreType.DMA,
        pltpu.SemaphoreType.DMA,
    ],
)(_sc_agg_body)


def _dot(a, b):
    return jnp.dot(a, b, preferred_element_type=jnp.float32)


def _hdot(a, b):
    return jnp.dot(a, b, preferred_element_type=jnp.float32, precision=_HIGH)


def _embed_body(x_ref, emb_ref, h_ref, hr_ref):
    oh = (x_ref[...] == lax.broadcasted_iota(jnp.int32, (1, 32), 1))
    h = _hdot(oh.astype(jnp.float32), emb_ref[...])
    h_ref[...] = h
    hr_ref[...] = jnp.maximum(h, 0.0)


_embed = pl.pallas_call(
    _embed_body,
    grid=(NB,),
    in_specs=[pl.BlockSpec((B, 1), lambda i: (i, 0)),
              pl.BlockSpec((32, D), lambda i: (0, 0))],
    out_specs=(pl.BlockSpec((B, D), lambda i: (i, 0)),
               pl.BlockSpec((B, D), lambda i: (i, 0))),
    out_shape=(jax.ShapeDtypeStruct((N, D), jnp.float32),
               jax.ShapeDtypeStruct((N, D), jnp.float32)),
)


def _k1_body(h_ref, a0_ref, a1_ref, eps_ref, w1_ref, b1_ref, z1_ref, st_ref):
    z = (1.0 + eps_ref[...]) * h_ref[...] + a0_ref[...] + a1_ref[...]
    z1 = _dot(z, w1_ref[...]) + b1_ref[...]
    z1_ref[...] = z1

    @pl.when(pl.program_id(0) == 0)
    def _():
        st_ref[...] = jnp.zeros_like(st_ref)

    s1 = jnp.sum(z1, axis=0, keepdims=True)
    s2 = jnp.sum(z1 * z1, axis=0, keepdims=True)
    st_ref[...] += jnp.concatenate([s1, s2], axis=0)


def _make_k1(h2d):
    return pl.pallas_call(
        _k1_body,
        grid=(NB,),
        in_specs=[pl.BlockSpec((B, D), lambda i: (i, 0)),
                  pl.BlockSpec((B, D), lambda i: (i, 0)),
                  pl.BlockSpec((B, D), lambda i: (i, 0)),
                  pl.BlockSpec((1, 1), lambda i: (0, 0)),
                  pl.BlockSpec((D, h2d), lambda i: (0, 0)),
                  pl.BlockSpec((1, h2d), lambda i: (0, 0))],
        out_specs=(pl.BlockSpec((B, h2d), lambda i: (i, 0)),
                   pl.BlockSpec((2, h2d), lambda i: (0, 0))),
        out_shape=(jax.ShapeDtypeStruct((N, h2d), jnp.float32),
                   jax.ShapeDtypeStruct((2, h2d), jnp.float32)),
    )


_k1 = _make_k1(2 * D)


def _norm(z, st_ref, gamma, beta):
    m = st_ref[0:1, :] * (1.0 / N)
    v = st_ref[1:2, :] * (1.0 / N) - m * m
    return (z - m) / jnp.sqrt(v + 1e-5) * gamma + beta


def _k2_body(z1_ref, st_ref, g1_ref, be1_ref, w2_ref, b2_ref, z2_ref, st2_ref):
    z = jnp.maximum(_norm(z1_ref[...], st_ref, g1_ref[...], be1_ref[...]), 0.0)
    z2 = _dot(z, w2_ref[...]) + b2_ref[...]
    z2_ref[...] = z2

    @pl.when(pl.program_id(0) == 0)
    def _():
        st2_ref[...] = jnp.zeros_like(st2_ref)

    s1 = jnp.sum(z2, axis=0, keepdims=True)
    s2 = jnp.sum(z2 * z2, axis=0, keepdims=True)
    st2_ref[...] += jnp.concatenate([s1, s2], axis=0)


_k2 = pl.pallas_call(
    _k2_body,
    grid=(NB,),
    in_specs=[pl.BlockSpec((B, 2 * D), lambda i: (i, 0)),
              pl.BlockSpec((2, 2 * D), lambda i: (0, 0)),
              pl.BlockSpec((1, 2 * D), lambda i: (0, 0)),
              pl.BlockSpec((1, 2 * D), lambda i: (0, 0)),
              pl.BlockSpec((2 * D, D), lambda i: (0, 0)),
              pl.BlockSpec((1, D), lambda i: (0, 0))],
    out_specs=(pl.BlockSpec((B, D), lambda i: (i, 0)),
               pl.BlockSpec((2, D), lambda i: (0, 0))),
    out_shape=(jax.ShapeDtypeStruct((N, D), jnp.float32),
               jax.ShapeDtypeStruct((2, D), jnp.float32)),
)


def _k3_body_relu(z2_ref, st_ref, gbn_ref, bbn_ref, h_ref):
    h_ref[...] = jnp.maximum(
        _norm(z2_ref[...], st_ref, gbn_ref[...], bbn_ref[...]), 0.0)


_k3 = pl.pallas_call(
    _k3_body_relu,
    grid=(NB,),
    in_specs=[pl.BlockSpec((B, D), lambda i: (i, 0)),
              pl.BlockSpec((2, D), lambda i: (0, 0)),
              pl.BlockSpec((1, D), lambda i: (0, 0)),
              pl.BlockSpec((1, D), lambda i: (0, 0))],
    out_specs=pl.BlockSpec((B, D), lambda i: (i, 0)),
    out_shape=jax.ShapeDtypeStruct((N, D), jnp.float32),
)


def _pool_body(z2_ref, st_ref, gbn_ref, bbn_ref, batch_ref,
               wc1_ref, bc1_ref, wc2_ref, bc2_ref, wc3_ref, bc3_ref,
               out_ref, sums_ref, cnt_ref):
    h = _norm(z2_ref[...], st_ref, gbn_ref[...], bbn_ref[...])
    oh = (batch_ref[...] == lax.broadcasted_iota(jnp.int32, (1, G), 1))
    ohf = oh.astype(jnp.float32)
    i = pl.program_id(0)

    @pl.when(i == 0)
    def _():
        sums_ref[...] = jnp.zeros_like(sums_ref)
        cnt_ref[...] = jnp.zeros_like(cnt_ref)

    tdot = functools.partial(lax.dot_general,
                             dimension_numbers=(((0,), (0,)), ((), ())),
                             preferred_element_type=jnp.float32,
                             precision=_HIGH)
    sums_ref[...] += tdot(ohf, h)
    cnt_ref[...] += tdot(ohf, jnp.ones((B, 1), jnp.float32))

    @pl.when(i == NB - 1)
    def _():
        pooled = sums_ref[...] / jnp.maximum(cnt_ref[...], 1.0)
        o = jnp.maximum(_dot(pooled, wc1_ref[...]) + bc1_ref[...], 0.0)
        o = jnp.maximum(_dot(o, wc2_ref[...]) + bc2_ref[...], 0.0)
        out_ref[...] = _dot(o, wc3_ref[...]) + bc3_ref[...]


_pool = pl.pallas_call(
    _pool_body,
    grid=(NB,),
    in_specs=[pl.BlockSpec((B, D), lambda i: (i, 0)),
              pl.BlockSpec((2, D), lambda i: (0, 0)),
              pl.BlockSpec((1, D), lambda i: (0, 0)),
              pl.BlockSpec((1, D), lambda i: (0, 0)),
              pl.BlockSpec((B, 1), lambda i: (i, 0)),
              pl.BlockSpec((D, D // 2), lambda i: (0, 0)),
              pl.BlockSpec((1, D // 2), lambda i: (0, 0)),
              pl.BlockSpec((D // 2, D // 4), lambda i: (0, 0)),
              pl.BlockSpec((1, D // 4), lambda i: (0, 0)),
              pl.BlockSpec((D // 4, C), lambda i: (0, 0)),
              pl.BlockSpec((1, C), lambda i: (0, 0))],
    out_specs=pl.BlockSpec((G, C), lambda i: (0, 0)),
    out_shape=jax.ShapeDtypeStruct((G, C), jnp.float32),
    scratch_shapes=[pltpu.VMEM((G, D), jnp.float32),
                    pltpu.VMEM((G, 1), jnp.float32)],
)


def kernel(x, edge_index, batch, ptr, emb, W1, b1, g1, be1, W2, b2, eps,
           gbn, bbn, Wc1, bc1, Wc2, bc2, Wc3, bc3):
    del ptr
    row = edge_index[0].astype(jnp.int32)
    col = edge_index[1].astype(jnp.int32)
    row_p = jnp.concatenate([row, jnp.zeros((EPAD,), jnp.int32)])
    col_p = jnp.concatenate([col, jnp.zeros((EPAD,), jnp.int32)])
    emb_p = jnp.zeros((32, D), jnp.float32).at[: emb.shape[0]].set(emb)
    zer = jnp.zeros((STRIPE, D), jnp.float32)
    x2 = x.astype(jnp.int32).reshape(N, 1)
    batch2 = batch.astype(jnp.int32).reshape(N, 1)

    h, hr = _embed(x2, emb_p)
    for l in range(L):
        agg0, agg1 = _sc_agg(hr, row_p, col_p, zer)
        z1, st1 = _k1(h, agg0, agg1, eps[l].reshape(1, 1), W1[l],
                      b1[l].reshape(1, 2 * D))
        z2, st2 = _k2(z1, st1, g1[l].reshape(1, 2 * D),
                      be1[l].reshape(1, 2 * D), W2[l], b2[l].reshape(1, D))
        gb = gbn[l].reshape(1, D)
        bb = bbn[l].reshape(1, D)
        if l < L - 1:
            h = _k3(z2, st2, gb, bb)
            hr = h
        else:
            out = _pool(z2, st2, gb, bb, batch2,
                        Wc1, bc1.reshape(1, D // 2),
                        Wc2, bc2.reshape(1, D // 4),
                        Wc3, bc3.reshape(1, C))
    return out
